# baseline (speedup 1.0000x reference)
import functools

import jax
import jax.numpy as jnp
from jax import lax
from jax.experimental import pallas as pl
from jax.experimental.pallas import tpu as pltpu
from jax.experimental.pallas import tpu_sc as plsc

_N = 10000
_NP = 10240
_E = 320000
_EP = 327680
_EC = 2560
_TC = 160
_K = 5
_ROWS_PER_TILE = _NP // 16

_MESH = dict(core_axis_name="c", subcore_axis_name="s", num_cores=2,
             num_subcores=16)
_RING = 3


def _zero_buf(buf, rows, width):
    def body(e, carry):
        for g in range(width // 16):
            buf[e, pl.ds(g * 16, 16)] = jnp.zeros((16,), jnp.float32)
        return carry
    lax.fori_loop(0, rows, body, 0)


_EPT = _EP // 16


def _precompute_body(row_hbm, ew_hbm, norm_hbm, row_v, ew_v, zbuf, dbuf,
                     nbuf, sem, acc):
    c = lax.axis_index("c")
    s = lax.axis_index("s")
    pltpu.sync_copy(row_hbm.at[pl.ds(s * _TC, _TC)], row_v)
    pltpu.sync_copy(ew_hbm.at[pl.ds(s * _EPT, _EPT)], ew_v)

    def zb(i, carry):
        zbuf[pl.ds(i * 16, 16)] = jnp.zeros((16,), jnp.float32)
        return carry
    lax.fori_loop(0, _ROWS_PER_TILE // 16, zb, 0)
    pltpu.sync_copy(zbuf, acc.at[pl.ds(s * _ROWS_PER_TILE, _ROWS_PER_TILE)])
    plsc.subcore_barrier()

    def chunk_deg(j, carry):
        pltpu.sync_copy(ew_v.at[pl.ds(j * 128, 128)],
                        acc.at[row_v.at[j]], add=True)
        return carry
    lax.fori_loop(0, _TC, chunk_deg, 0)
    plsc.subcore_barrier()

    def chunk_norm(j, carry):
        pltpu.sync_copy(acc.at[row_v.at[j]], dbuf)
        def grp(g, carry2):
            sl = pl.ds(j * 128 + g * 16, 16)
            dv = dbuf[pl.ds(g * 16, 16)]
            ewg = ew_v[sl]
            sel = dv > 0.0
            safe = jnp.where(sel, dv, 1.0)
            nbuf[sl] = jnp.where(sel, -(ewg / safe), 0.0)
            return carry2
        lax.fori_loop(0, 8, grp, 0)
        return carry
    lax.fori_loop(0, _TC, chunk_norm, 0)

    half = _EPT // 2
    pltpu.sync_copy(nbuf.at[pl.ds(c * half, half)],
                    norm_hbm.at[pl.ds(s * _EPT + c * half, half)])


@jax.jit
def _precompute_norm(row2, ew1):
    return pl.kernel(
        _precompute_body,
        out_type=jax.ShapeDtypeStruct((_EP,), jnp.float32),
        mesh=plsc.VectorSubcoreMesh(**_MESH),
        scratch_types=[
            pltpu.VMEM((_TC, 128), jnp.int32),
            pltpu.VMEM((_EPT,), jnp.float32),
            pltpu.VMEM((_ROWS_PER_TILE,), jnp.float32),
            pltpu.VMEM((128,), jnp.float32),
            pltpu.VMEM((_EPT,), jnp.float32),
            pltpu.SemaphoreType.DMA,
            pltpu.VMEM_SHARED((_NP,), jnp.float32),
        ],
        compiler_params=pltpu.CompilerParams(use_tc_tiling_on_sc=False),
    )(row2, ew1)


def _prop_body(h_hbm, norm_hbm, col_hbm, row_hbm, *args, d2p, has_prev):
    ring = _RING
    if has_prev:
        prev_hbm, rest = args[0], args[1:]
    else:
        prev_hbm, rest = None, args
    out_hbm, row_v, col_v = rest[0:3]
    bufs = rest[3:3 + ring]
    nbufs = rest[3 + ring:3 + 2 * ring]
    sems = rest[3 + 2 * ring:3 + 5 * ring]
    acc = rest[-1]
    gsems = sems[0:ring]
    ssems = sems[ring:2 * ring]
    nsems = sems[2 * ring:3 * ring]
    tmpa, tmpb = bufs[0], bufs[1]
    c = lax.axis_index("c")
    s = lax.axis_index("s")
    pltpu.sync_copy(row_hbm.at[pl.ds(s * _TC, _TC)], row_v)
    pltpu.sync_copy(col_hbm.at[pl.ds(s * _TC, _TC)], col_v)
    nbase = s * _EPT

    off = c * _NP
    def adj(j, carry):
        for g in range(8):
            sl = pl.ds(g * 16, 16)
            row_v[j, sl] = row_v[j, sl] + off
        return carry
    lax.fori_loop(0, _TC, adj, 0)

    _zero_buf(tmpa, 128, d2p)
    for b in range(_ROWS_PER_TILE // 128):
        pltpu.sync_copy(tmpa, acc.at[pl.ds((s * 5 + b) * 128, 128)])
    plsc.subcore_barrier()

    for b in range(2):
        pltpu.async_copy(h_hbm.at[row_v.at[b]], bufs[b], gsems[b])
        pltpu.async_copy(norm_hbm.at[pl.ds(nbase + b * 128, 128)],
                         nbufs[b], nsems[b])

    def step(j, b):
        gb = bufs[b]
        nb = (b + 2) % ring
        pltpu.make_async_copy(h_hbm.at[row_v.at[j]], gb, gsems[b]).wait()
        pltpu.make_async_copy(norm_hbm.at[pl.ds(nbase + j * 128, 128)],
                              nbufs[b], nsems[b]).wait()
        def grp(g, carry2):
            nv16 = nbufs[b][pl.ds(g * 16, 16)]
            e0 = g * 16
            for l in range(16):
                nv = nv16[l]
                for gg in range(d2p // 16):
                    sl = pl.ds(gg * 16, 16)
                    gb[e0 + l, sl] = gb[e0 + l, sl] * nv
            return carry2
        lax.fori_loop(0, 8, grp, 0)
        pltpu.async_copy(gb, acc.at[col_v.at[j]], ssems[b], add=True)
        def _wait_old_scatter():
            pltpu.make_async_copy(bufs[nb], acc.at[col_v.at[j - (ring - 2)]],
                                  ssems[nb]).wait()
        def _next_gather():
            pltpu.async_copy(h_hbm.at[row_v.at[j + 2]], bufs[nb],
                             gsems[nb])
            pltpu.async_copy(norm_hbm.at[pl.ds(nbase + (j + 2) * 128, 128)],
                             nbufs[nb], nsems[nb])
        pl.when(j >= ring - 2)(_wait_old_scatter)
        pl.when(j + 2 < _TC)(_next_gather)

    n_main = (_TC - 4) // ring * ring
    def outer(t, carry):
        for b in range(ring):
            step(ring * t + b, b)
        return carry
    lax.fori_loop(0, n_main // ring, outer, 0)
    for j in range(n_main, _TC):
        step(j, j % ring)
    for j in range(_TC - (ring - 2), _TC):
        pltpu.make_async_copy(bufs[j % ring], acc.at[col_v.at[j]],
                              ssems[j % ring]).wait()
    plsc.subcore_barrier()

    for b in range(_ROWS_PER_TILE // 128):
        base = (s * 5 + b) * 128
        pltpu.sync_copy(acc.at[pl.ds(base, 128)], tmpa)
        if has_prev:
            pltpu.sync_copy(prev_hbm.at[pl.ds(off + base, 128)], tmpb)
            def fold(e, carry):
                for g in range(d2p // 16):
                    sl = pl.ds(g * 16, 16)
                    tmpa[e, sl] = 2.0 * tmpa[e, sl] - tmpb[e, sl]
                return carry
            lax.fori_loop(0, 128, fold, 0)
        pltpu.sync_copy(tmpa, out_hbm.at[pl.ds(off + base, 128)])


@functools.partial(jax.jit, static_argnames=("d2p", "has_prev"))
def _prop(h_flat, norm2, col2, row2, prev_flat=None, *, d2p, has_prev):
    scratch = [
        pltpu.VMEM((_TC, 128), jnp.int32),
        pltpu.VMEM((_TC, 128), jnp.int32),
    ] + [pltpu.VMEM((128, d2p), jnp.float32) for _ in range(_RING)] \
      + [pltpu.VMEM((128,), jnp.float32) for _ in range(_RING)] \
      + [pltpu.SemaphoreType.DMA] * (3 * _RING) + [
        pltpu.VMEM_SHARED((_NP, d2p), jnp.float32),
    ]
    fn = pl.kernel(
        functools.partial(_prop_body, d2p=d2p, has_prev=has_prev),
        out_type=jax.ShapeDtypeStruct((2 * _NP, d2p), jnp.float32),
        mesh=plsc.VectorSubcoreMesh(**_MESH),
        scratch_types=scratch,
        compiler_params=pltpu.CompilerParams(use_tc_tiling_on_sc=False),
    )
    if has_prev:
        return fn(h_flat, norm2, col2, row2, prev_flat)
    return fn(h_flat, norm2, col2, row2)


def _combine_body(*refs, act, s_in, s_out, d2p_out, half_out):
    nx = _K * s_in
    x_refs = refs[:nx]
    w_ref = refs[nx]
    b_ref = refs[nx + 1]
    o_refs = refs[nx + 2:]
    acc = None
    for k in range(_K):
        for si in range(s_in):
            xr = x_refs[k * s_in + si]
            for c in range(2):
                p = jnp.dot(xr[c], w_ref[k, si, c],
                            preferred_element_type=jnp.float32)
                acc = p if acc is None else acc + p
    z = acc + b_ref[0]
    if act:
        z = jax.nn.softplus(z)
    tn = z.shape[0]
    for si in range(s_out):
        for c in range(2):
            col0 = (si * 2 + c) * half_out
            piece = z[:, col0:col0 + half_out]
            if d2p_out > half_out:
                padz = jnp.zeros((tn, d2p_out - half_out), jnp.float32)
                piece = jnp.concatenate([piece, padz], axis=1)
            o_refs[si][c] = piece


def _combine(txs, w_s, b, act, s_out, d2p_out, half_out):
    d2p_in = txs[0].shape[-1]
    s_in = len(txs) // _K
    dout = w_s.shape[-1]
    tn = 640
    grid = (_NP // tn,)
    body = functools.partial(_combine_body, act=act, s_in=s_in, s_out=s_out,
                             d2p_out=d2p_out, half_out=half_out)
    return pl.pallas_call(
        body,
        grid=grid,
        in_specs=[pl.BlockSpec((2, tn, d2p_in), lambda i: (0, i, 0))
                  for _ in txs] + [
            pl.BlockSpec(w_s.shape, lambda i: (0,) * 5),
            pl.BlockSpec((1, dout), lambda i: (0, 0)),
        ],
        out_specs=[pl.BlockSpec((2, tn, d2p_out), lambda i: (0, i, 0))
                   for _ in range(s_out)],
        out_shape=[jax.ShapeDtypeStruct((2, _NP, d2p_out), jnp.float32)
                   for _ in range(s_out)],
    )(*txs, w_s, b[None, :])


def _fc_body(*refs):
    x_refs = refs[:-3]
    w_ref, b_ref, o_ref = refs[-3], refs[-2], refs[-1]
    acc = None
    for si in range(len(x_refs)):
        for c in range(2):
            p = jnp.dot(x_refs[si][c], w_ref[si, c],
                        preferred_element_type=jnp.float32)
            acc = p if acc is None else acc + p
    o_ref[...] = acc + b_ref[0]


def _fc(hs, w_s, b):
    tn = 1000
    grid = (_N // tn,)
    d2p = hs[0].shape[-1]
    nc = w_s.shape[-1]
    return pl.pallas_call(
        _fc_body,
        grid=grid,
        in_specs=[pl.BlockSpec((2, tn, d2p), lambda i: (0, i, 0))
                  for _ in hs] + [
            pl.BlockSpec(w_s.shape, lambda i: (0,) * 4),
            pl.BlockSpec((1, nc), lambda i: (0, 0)),
        ],
        out_specs=pl.BlockSpec((tn, nc), lambda i: (i, 0)),
        out_shape=jax.ShapeDtypeStruct((_N, nc), jnp.float32),
    )(*hs, w_s, b[None, :])


_DIMS = [(128, 16), (16, 32), (32, 64), (64, 64), (64, 128), (128, 256),
         (256, 512)]


def _split(d):
    s = max(d // 128, 1)
    half = d // (2 * s)
    d2p = max(half, 16)
    return s, half, d2p


def kernel(x, edge_weigth, params, edge_index, batch):
    row = edge_index[0]
    col = edge_index[1]
    row2 = jnp.pad(row, (0, _EP - _E)).reshape(_EC, 128)
    col2 = jnp.pad(col, (0, _EP - _E)).reshape(_EC, 128)
    ew1 = jnp.pad(edge_weigth, (0, _EP - _E))

    norm2 = _precompute_norm(row2, ew1)

    xp = jnp.pad(x, ((0, _NP - _N), (0, 0)))
    hs = [xp.reshape(_NP, 2, 64).transpose(1, 0, 2).reshape(2 * _NP, 64)]

    for i, (din, dout) in enumerate(_DIMS):
        s_in, half_in, d2i = _split(din)
        w = params["W%d" % i]
        b = params["b%d" % i]
        w2 = w.reshape(_K, s_in, 2, half_in, dout)
        if d2i > half_in:
            w2 = jnp.pad(w2, ((0, 0), (0, 0), (0, 0), (0, d2i - half_in),
                              (0, 0)))
        def p(h, prev=None):
            return _prop(h, norm2, col2, row2, prev, d2p=d2i,
                         has_prev=prev is not None)
        tx0 = hs
        tx1 = [p(t) for t in tx0]
        tx2 = [p(a, b2) for a, b2 in zip(tx1, tx0)]
        tx3 = [p(a, b2) for a, b2 in zip(tx2, tx1)]
        tx4 = [p(a, b2) for a, b2 in zip(tx3, tx2)]
        txs = [t.reshape(2, _NP, d2i)
               for tk in (tx0, tx1, tx2, tx3, tx4) for t in tk]
        s_out, half_out, d2o = _split(dout)
        h3ds = _combine(txs, w2, b, act=True, s_out=s_out, d2p_out=d2o,
                        half_out=half_out)
        hs = [t.reshape(2 * _NP, d2o) for t in h3ds]

    s7, half7, d2p7 = _split(512)
    fcw = params["fc_w"].T.reshape(s7, 2, half7, -1)
    return _fc([t.reshape(2, _NP, d2p7) for t in hs], fcw, params["fc_b"])


# speedup vs baseline: 1.2331x; 1.2331x over previous
import functools

import jax
import jax.numpy as jnp
from jax import lax
from jax.experimental import pallas as pl
from jax.experimental.pallas import tpu as pltpu
from jax.experimental.pallas import tpu_sc as plsc

_N = 10000
_NP = 10240
_E = 320000
_EP = 327680
_EC = 2560
_TC = 160
_K = 5
_ROWS_PER_TILE = _NP // 16

_MESH = dict(core_axis_name="c", subcore_axis_name="s", num_cores=2,
             num_subcores=16)
_RING = 3


def _zero_buf(buf, rows, width):
    def body(e, carry):
        for g in range(width // 16):
            buf[e, pl.ds(g * 16, 16)] = jnp.zeros((16,), jnp.float32)
        return carry
    lax.fori_loop(0, rows, body, 0)


_EPT = _EP // 16


def _precompute_body(row_hbm, ew_hbm, norm_hbm, row_v, ew_v, zbuf, dbuf,
                     nbuf, sem, acc):
    c = lax.axis_index("c")
    s = lax.axis_index("s")
    pltpu.sync_copy(row_hbm.at[pl.ds(s * _TC, _TC)], row_v)
    pltpu.sync_copy(ew_hbm.at[pl.ds(s * _EPT, _EPT)], ew_v)

    def zb(i, carry):
        zbuf[pl.ds(i * 16, 16)] = jnp.zeros((16,), jnp.float32)
        return carry
    lax.fori_loop(0, _ROWS_PER_TILE // 16, zb, 0)
    pltpu.sync_copy(zbuf, acc.at[pl.ds(s * _ROWS_PER_TILE, _ROWS_PER_TILE)])
    plsc.subcore_barrier()

    def chunk_deg(j, carry):
        pltpu.sync_copy(ew_v.at[pl.ds(j * 128, 128)],
                        acc.at[row_v.at[j]], add=True)
        return carry
    lax.fori_loop(0, _TC, chunk_deg, 0)
    plsc.subcore_barrier()

    def chunk_norm(j, carry):
        pltpu.sync_copy(acc.at[row_v.at[j]], dbuf)
        def grp(g, carry2):
            sl = pl.ds(j * 128 + g * 16, 16)
            dv = dbuf[pl.ds(g * 16, 16)]
            ewg = ew_v[sl]
            sel = dv > 0.0
            safe = jnp.where(sel, dv, 1.0)
            nbuf[sl] = jnp.where(sel, -(ewg / safe), 0.0)
            return carry2
        lax.fori_loop(0, 8, grp, 0)
        return carry
    lax.fori_loop(0, _TC, chunk_norm, 0)

    half = _EPT // 2
    pltpu.sync_copy(nbuf.at[pl.ds(c * half, half)],
                    norm_hbm.at[pl.ds(s * _EPT + c * half, half)])


@jax.jit
def _precompute_norm(row2, ew1):
    return pl.kernel(
        _precompute_body,
        out_type=jax.ShapeDtypeStruct((_EP,), jnp.float32),
        mesh=plsc.VectorSubcoreMesh(**_MESH),
        scratch_types=[
            pltpu.VMEM((_TC, 128), jnp.int32),
            pltpu.VMEM((_EPT,), jnp.float32),
            pltpu.VMEM((_ROWS_PER_TILE,), jnp.float32),
            pltpu.VMEM((128,), jnp.float32),
            pltpu.VMEM((_EPT,), jnp.float32),
            pltpu.SemaphoreType.DMA,
            pltpu.VMEM_SHARED((_NP,), jnp.float32),
        ],
        compiler_params=pltpu.CompilerParams(use_tc_tiling_on_sc=False),
    )(row2, ew1)


def _prop_body(h_hbm, norm_hbm, col_hbm, row_hbm, *args, d2p, has_prev):
    ring = _RING
    if has_prev:
        prev_hbm, rest = args[0], args[1:]
    else:
        prev_hbm, rest = None, args
    out_hbm, row_v, col_v, nrm_v = rest[0:4]
    bufs = rest[4:4 + ring]
    sems = rest[4 + ring:4 + 3 * ring]
    acc = rest[-1]
    gsems = sems[0:ring]
    ssems = sems[ring:2 * ring]
    tmpa, tmpb = bufs[0], bufs[1]
    c = lax.axis_index("c")
    s = lax.axis_index("s")
    pltpu.sync_copy(row_hbm.at[pl.ds(s * _TC, _TC)], row_v)
    pltpu.sync_copy(col_hbm.at[pl.ds(s * _TC, _TC)], col_v)
    pltpu.sync_copy(norm_hbm.at[pl.ds(s * _EPT, _EPT)], nrm_v)

    off = c * _NP
    def adj(j, carry):
        for g in range(8):
            sl = pl.ds(g * 16, 16)
            row_v[j, sl] = row_v[j, sl] + off
        return carry
    lax.fori_loop(0, _TC, adj, 0)

    _zero_buf(tmpa, 128, d2p)
    for b in range(_ROWS_PER_TILE // 128):
        pltpu.sync_copy(tmpa, acc.at[pl.ds((s * 5 + b) * 128, 128)])
    plsc.subcore_barrier()

    for b in range(2):
        pltpu.async_copy(h_hbm.at[row_v.at[b]], bufs[b], gsems[b])

    def step(j, b):
        gb = bufs[b]
        nb = (b + 2) % ring
        pltpu.make_async_copy(h_hbm.at[row_v.at[j]], gb, gsems[b]).wait()
        jbase = j * 128
        def grp(g, carry2):
            nv16 = nrm_v[pl.ds(jbase + g * 16, 16)]
            e0 = g * 16
            for l in range(16):
                nv = nv16[l]
                for gg in range(d2p // 16):
                    sl = pl.ds(gg * 16, 16)
                    gb[e0 + l, sl] = gb[e0 + l, sl] * nv
            return carry2
        lax.fori_loop(0, 8, grp, 0)
        pltpu.async_copy(gb, acc.at[col_v.at[j]], ssems[b], add=True)
        def _wait_old_scatter():
            pltpu.make_async_copy(bufs[nb], acc.at[col_v.at[j - (ring - 2)]],
                                  ssems[nb]).wait()
        def _next_gather():
            pltpu.async_copy(h_hbm.at[row_v.at[j + 2]], bufs[nb],
                             gsems[nb])
        pl.when(j >= ring - 2)(_wait_old_scatter)
        pl.when(j + 2 < _TC)(_next_gather)

    n_main = (_TC - 4) // ring * ring
    def outer(t, carry):
        for b in range(ring):
            step(ring * t + b, b)
        return carry
    lax.fori_loop(0, n_main // ring, outer, 0)
    for j in range(n_main, _TC):
        step(j, j % ring)
    for j in range(_TC - (ring - 2), _TC):
        pltpu.make_async_copy(bufs[j % ring], acc.at[col_v.at[j]],
                              ssems[j % ring]).wait()
    plsc.subcore_barrier()

    for b in range(_ROWS_PER_TILE // 128):
        base = (s * 5 + b) * 128
        pltpu.sync_copy(acc.at[pl.ds(base, 128)], tmpa)
        if has_prev:
            pltpu.sync_copy(prev_hbm.at[pl.ds(off + base, 128)], tmpb)
            def fold(e, carry):
                for g in range(d2p // 16):
                    sl = pl.ds(g * 16, 16)
                    tmpa[e, sl] = 2.0 * tmpa[e, sl] - tmpb[e, sl]
                return carry
            lax.fori_loop(0, 128, fold, 0)
        pltpu.sync_copy(tmpa, out_hbm.at[pl.ds(off + base, 128)])


@functools.partial(jax.jit, static_argnames=("d2p", "has_prev"))
def _prop(h_flat, norm2, col2, row2, prev_flat=None, *, d2p, has_prev):
    scratch = [
        pltpu.VMEM((_TC, 128), jnp.int32),
        pltpu.VMEM((_TC, 128), jnp.int32),
        pltpu.VMEM((_EPT,), jnp.float32),
    ] + [pltpu.VMEM((128, d2p), jnp.float32) for _ in range(_RING)] \
      + [pltpu.SemaphoreType.DMA] * (2 * _RING) + [
        pltpu.VMEM_SHARED((_NP, d2p), jnp.float32),
    ]
    fn = pl.kernel(
        functools.partial(_prop_body, d2p=d2p, has_prev=has_prev),
        out_type=jax.ShapeDtypeStruct((2 * _NP, d2p), jnp.float32),
        mesh=plsc.VectorSubcoreMesh(**_MESH),
        scratch_types=scratch,
        compiler_params=pltpu.CompilerParams(use_tc_tiling_on_sc=False),
    )
    if has_prev:
        return fn(h_flat, norm2, col2, row2, prev_flat)
    return fn(h_flat, norm2, col2, row2)


def _combine_body(*refs, act, s_in, s_out, d2p_out, half_out):
    nx = _K * s_in
    x_refs = refs[:nx]
    w_ref = refs[nx]
    b_ref = refs[nx + 1]
    o_refs = refs[nx + 2:]
    acc = None
    for k in range(_K):
        for si in range(s_in):
            xr = x_refs[k * s_in + si]
            for c in range(2):
                p = jnp.dot(xr[c], w_ref[k, si, c],
                            preferred_element_type=jnp.float32)
                acc = p if acc is None else acc + p
    z = acc + b_ref[0]
    if act:
        z = jax.nn.softplus(z)
    tn = z.shape[0]
    for si in range(s_out):
        for c in range(2):
            col0 = (si * 2 + c) * half_out
            piece = z[:, col0:col0 + half_out]
            if d2p_out > half_out:
                padz = jnp.zeros((tn, d2p_out - half_out), jnp.float32)
                piece = jnp.concatenate([piece, padz], axis=1)
            o_refs[si][c] = piece


def _combine(txs, w_s, b, act, s_out, d2p_out, half_out):
    d2p_in = txs[0].shape[-1]
    s_in = len(txs) // _K
    dout = w_s.shape[-1]
    tn = 640
    grid = (_NP // tn,)
    body = functools.partial(_combine_body, act=act, s_in=s_in, s_out=s_out,
                             d2p_out=d2p_out, half_out=half_out)
    return pl.pallas_call(
        body,
        grid=grid,
        in_specs=[pl.BlockSpec((2, tn, d2p_in), lambda i: (0, i, 0))
                  for _ in txs] + [
            pl.BlockSpec(w_s.shape, lambda i: (0,) * 5),
            pl.BlockSpec((1, dout), lambda i: (0, 0)),
        ],
        out_specs=[pl.BlockSpec((2, tn, d2p_out), lambda i: (0, i, 0))
                   for _ in range(s_out)],
        out_shape=[jax.ShapeDtypeStruct((2, _NP, d2p_out), jnp.float32)
                   for _ in range(s_out)],
    )(*txs, w_s, b[None, :])


def _fc_body(*refs):
    x_refs = refs[:-3]
    w_ref, b_ref, o_ref = refs[-3], refs[-2], refs[-1]
    acc = None
    for si in range(len(x_refs)):
        for c in range(2):
            p = jnp.dot(x_refs[si][c], w_ref[si, c],
                        preferred_element_type=jnp.float32)
            acc = p if acc is None else acc + p
    o_ref[...] = acc + b_ref[0]


def _fc(hs, w_s, b):
    tn = 1000
    grid = (_N // tn,)
    d2p = hs[0].shape[-1]
    nc = w_s.shape[-1]
    return pl.pallas_call(
        _fc_body,
        grid=grid,
        in_specs=[pl.BlockSpec((2, tn, d2p), lambda i: (0, i, 0))
                  for _ in hs] + [
            pl.BlockSpec(w_s.shape, lambda i: (0,) * 4),
            pl.BlockSpec((1, nc), lambda i: (0, 0)),
        ],
        out_specs=pl.BlockSpec((tn, nc), lambda i: (i, 0)),
        out_shape=jax.ShapeDtypeStruct((_N, nc), jnp.float32),
    )(*hs, w_s, b[None, :])


_DIMS = [(128, 16), (16, 32), (32, 64), (64, 64), (64, 128), (128, 256),
         (256, 512)]


def _split(d):
    s = max(d // 128, 1)
    half = d // (2 * s)
    d2p = max(half, 16)
    return s, half, d2p


def kernel(x, edge_weigth, params, edge_index, batch):
    row = edge_index[0]
    col = edge_index[1]
    row2 = jnp.pad(row, (0, _EP - _E)).reshape(_EC, 128)
    col2 = jnp.pad(col, (0, _EP - _E)).reshape(_EC, 128)
    ew1 = jnp.pad(edge_weigth, (0, _EP - _E))

    norm2 = _precompute_norm(row2, ew1)

    xp = jnp.pad(x, ((0, _NP - _N), (0, 0)))
    hs = [xp.reshape(_NP, 2, 64).transpose(1, 0, 2).reshape(2 * _NP, 64)]

    for i, (din, dout) in enumerate(_DIMS):
        s_in, half_in, d2i = _split(din)
        w = params["W%d" % i]
        b = params["b%d" % i]
        w2 = w.reshape(_K, s_in, 2, half_in, dout)
        if d2i > half_in:
            w2 = jnp.pad(w2, ((0, 0), (0, 0), (0, 0), (0, d2i - half_in),
                              (0, 0)))
        def p(h, prev=None):
            return _prop(h, norm2, col2, row2, prev, d2p=d2i,
                         has_prev=prev is not None)
        tx0 = hs
        tx1 = [p(t) for t in tx0]
        tx2 = [p(a, b2) for a, b2 in zip(tx1, tx0)]
        tx3 = [p(a, b2) for a, b2 in zip(tx2, tx1)]
        tx4 = [p(a, b2) for a, b2 in zip(tx3, tx2)]
        txs = [t.reshape(2, _NP, d2i)
               for tk in (tx0, tx1, tx2, tx3, tx4) for t in tk]
        s_out, half_out, d2o = _split(dout)
        h3ds = _combine(txs, w2, b, act=True, s_out=s_out, d2p_out=d2o,
                        half_out=half_out)
        hs = [t.reshape(2 * _NP, d2o) for t in h3ds]

    s7, half7, d2p7 = _split(512)
    fcw = params["fc_w"].T.reshape(s7, 2, half7, -1)
    return _fc([t.reshape(2, _NP, d2p7) for t in hs], fcw, params["fc_b"])
